# baseline (speedup 1.0000x reference)
import functools

import jax
import jax.numpy as jnp
from jax import lax
from jax.experimental import pallas as pl
from jax.experimental.pallas import tpu as pltpu
from jax.experimental.pallas import tpu_sc as plsc

NUM_WORKERS = 32


@jax.jit
def _gather(idx, table):
    b, l_dim = idx.shape
    v, d = table.shape
    bpw = b // NUM_WORKERS
    ci_dim = d // 8

    mesh = plsc.VectorSubcoreMesh(core_axis_name="c", subcore_axis_name="s")

    @functools.partial(
        pl.kernel,
        out_type=jax.ShapeDtypeStruct((l_dim, ci_dim, NUM_WORKERS, 8, 128), jnp.float32),
        mesh=mesh,
        scratch_types=[
            pltpu.VMEM((bpw, l_dim), jnp.int32),
            pltpu.VMEM((2, bpw), jnp.int32),
            pltpu.VMEM((2, bpw, d), jnp.float32),
            pltpu.VMEM((2, ci_dim, 1, 8, bpw), jnp.float32),
            pltpu.SemaphoreType.DMA,
            pltpu.SemaphoreType.DMA,
            pltpu.SemaphoreType.DMA,
        ],
        compiler_params=pltpu.CompilerParams(
            use_tc_tiling_on_sc=False, needs_layout_passes=False
        ),
    )
    def k(idx_hbm, table_hbm, out_hbm, idx_v, idxcol, bufg, buft, gsem, osem0, osem1):
        wid = lax.axis_index("s") * 2 + lax.axis_index("c")
        b0 = wid * bpw
        pltpu.sync_copy(idx_hbm.at[pl.ds(b0, bpw)], idx_v)
        lanes = lax.iota(jnp.int32, 16)
        osems = (osem0, osem1)

        def build_idxcol(lidx, sel):
            lvec = jnp.full((16,), lidx, jnp.int32)

            def bg(g, c):
                rows = lanes + g * 16
                idxcol[sel, pl.ds(g * 16, 16)] = plsc.load_gather(idx_v, [rows, lvec])
                return c

            lax.fori_loop(0, bpw // 16, bg, 0)

        def fire_gather(sel):
            pltpu.async_copy(table_hbm.at[idxcol.at[sel]], bufg.at[sel], gsem)

        def wait_gather(sel):
            pltpu.make_async_copy(
                table_hbm.at[idxcol.at[sel]], bufg.at[sel], gsem
            ).wait()

        def transpose(sel):
            src = bufg.at[sel]
            dst = buft.at[sel]

            def tg(g, c):
                rows = lanes + g * 16
                for cc in range(d):
                    col = jnp.full((16,), cc, jnp.int32)
                    dst[cc // 8, 0, cc % 8, pl.ds(g * 16, 16)] = plsc.load_gather(
                        src, [rows, col]
                    )
                return c

            lax.fori_loop(0, bpw // 16, tg, 0)

        def out_slice(lidx):
            return out_hbm.at[lidx, pl.ds(0, ci_dim), pl.ds(wid, 1)]

        def fire_out(lidx, sel):
            pltpu.async_copy(buft.at[sel], out_slice(lidx), osems[sel])

        def wait_out(lidx, sel):
            pltpu.make_async_copy(buft.at[sel], out_slice(lidx), osems[sel]).wait()

        def step(lidx, sel, fire_next, wait_o):
            if fire_next:
                build_idxcol(lidx + 1, 1 - sel)
                fire_gather(1 - sel)
            wait_gather(sel)
            if wait_o:
                wait_out(lidx - 2, sel)
            transpose(sel)
            fire_out(lidx, sel)

        build_idxcol(0, 0)
        fire_gather(0)
        step(0, 0, True, False)
        step(1, 1, True, False)

        def body(i, c):
            step(2 * i + 2, 0, True, True)
            step(2 * i + 3, 1, True, True)
            return c

        lax.fori_loop(0, (l_dim - 4) // 2, body, 0)
        step(l_dim - 2, 0, True, True)
        step(l_dim - 1, 1, False, True)
        wait_out(l_dim - 2, 0)
        wait_out(l_dim - 1, 1)

    return k(idx, table)


def kernel(token_ids, W):
    b, l = token_ids.shape
    v, d = W.shape
    out5 = _gather(token_ids.astype(jnp.int32), W)
    y = jnp.transpose(out5, (2, 4, 0, 1, 3))
    return y.reshape(b, l, d)
